# baseline (device time: 18734 ns/iter reference)
import os

import jax
import jax.numpy as jnp
from jax import lax
from jax.experimental import pallas as pl
from jax.experimental.pallas import tpu as pltpu

_NOCOMM = os.environ.get("NOCOMM") == "1"

N_DEV = 8
B, SQ, D = 2, 128, 512
HQ, HKV, DH = 8, 2, 64
GROUP = HQ // HKV
SKV_SH = 128
LANES = HKV * DH
QR = B * SQ
SLICE = QR // N_DEV
PW = HQ * DH + LANES


def kernel(x, Wq, Wo, K_ext, V_ext):
    k3 = K_ext.reshape(B, SKV_SH, LANES)
    v3 = V_ext.reshape(B, SKV_SH, LANES)

    def body(x_ref, wq_hbm, wo_hbm, k_ref, v_ref, out_ref,
             part_s, pbuf, ogather, wq_vm, wo_vm,
             s1_sems, r1_sems, s2_sems, r2_sems, w_sems):
        my = lax.axis_index("i")

        wq_cp = pltpu.make_async_copy(wq_hbm, wq_vm, w_sems.at[0])
        wo_cp = pltpu.make_async_copy(wo_hbm, wo_vm, w_sems.at[1])
        wq_cp.start()
        wo_cp.start()

        barrier = pltpu.get_barrier_semaphore()
        for j in range(N_DEV - 1):
            peer = (my + 1 + j) % N_DEV
            pl.semaphore_signal(
                barrier, inc=1, device_id=(peer,),
                device_id_type=pl.DeviceIdType.MESH,
            )

        wq_cp.wait()
        wq_bf = wq_vm[:, :].astype(jnp.bfloat16)
        qs = []
        for b in range(B):
            qb = jnp.dot(x_ref[b].astype(jnp.bfloat16), wq_bf,
                         preferred_element_type=jnp.float32)
            qs.append(qb * (0.125 * 1.4426950408889634))

        o_parts, l_parts = [], []
        for b in range(B):
            for kh in range(HKV):
                qg = jnp.concatenate(
                    [qs[b][:, (kh * GROUP + g) * DH:(kh * GROUP + g + 1) * DH]
                     for g in range(GROUP)], axis=0).astype(jnp.bfloat16)
                kc = k_ref[b][:, kh * DH:(kh + 1) * DH].astype(jnp.bfloat16)
                vc = v_ref[b][:, kh * DH:(kh + 1) * DH].astype(jnp.bfloat16)
                s_mat = lax.dot_general(
                    qg, kc, (((1,), (1,)), ((), ())),
                    preferred_element_type=jnp.float32)
                p = jnp.exp2(s_mat)
                l_parts.append(jnp.sum(p, axis=1, keepdims=True))
                o_parts.append(lax.dot_general(
                    p.astype(jnp.bfloat16), vc, (((1,), (0,)), ((), ())),
                    preferred_element_type=jnp.float32))

        rows = []
        for b in range(B):
            ob = jnp.concatenate(
                [o_parts[b * HKV + kh][g * SQ:(g + 1) * SQ, :]
                 for kh in range(HKV) for g in range(GROUP)], axis=1)
            lb = jnp.concatenate(
                [l_parts[b * HKV + kh][g * SQ:(g + 1) * SQ, :]
                 for kh in range(HKV) for g in range(GROUP)]
                + [jnp.zeros((SQ, LANES - HQ), jnp.float32)], axis=1)
            rows.append(jnp.concatenate([ob, lb], axis=1))
        part_s[:, :] = jnp.concatenate(rows, axis=0).astype(jnp.bfloat16)

        pbuf[pl.ds(my * SLICE, SLICE), :] = part_s[pl.ds(my * SLICE, SLICE), :]

        pl.semaphore_wait(barrier, N_DEV - 1)

        sends = []
        for j in range(N_DEV - 1) if not _NOCOMM else []:
            peer = (my + 1 + j) % N_DEV
            rdma = pltpu.make_async_remote_copy(
                src_ref=part_s.at[pl.ds(peer * SLICE, SLICE)],
                dst_ref=pbuf.at[pl.ds(my * SLICE, SLICE)],
                send_sem=s1_sems.at[j],
                recv_sem=r1_sems.at[my],
                device_id=(peer,),
                device_id_type=pl.DeviceIdType.MESH,
            )
            rdma.start()
            sends.append(rdma)

        acc = pbuf[pl.ds(my * SLICE, SLICE), :].astype(jnp.float32)
        for j in range(N_DEV - 1) if not _NOCOMM else []:
            src = (my - 1 - j) % N_DEV
            recv = pltpu.make_async_remote_copy(
                src_ref=part_s.at[pl.ds(src * SLICE, SLICE)],
                dst_ref=pbuf.at[pl.ds(src * SLICE, SLICE)],
                send_sem=s1_sems.at[j],
                recv_sem=r1_sems.at[src],
                device_id=(my,),
                device_id_type=pl.DeviceIdType.MESH,
            )
            recv.wait_recv()
            acc = acc + pbuf[pl.ds(src * SLICE, SLICE), :].astype(jnp.float32)
        oh = jnp.concatenate(
            [acc[:, h * DH:(h + 1) * DH] / acc[:, HQ * DH + h:HQ * DH + h + 1]
             for h in range(HQ)], axis=1)
        wo_cp.wait()
        out_slice = jnp.dot(oh.astype(jnp.bfloat16),
                            wo_vm[:, :].astype(jnp.bfloat16),
                            preferred_element_type=jnp.float32)
        ogather[pl.ds(my * SLICE, SLICE), :] = out_slice.astype(jnp.bfloat16)

        for j in range(N_DEV - 1) if not _NOCOMM else []:
            peer = (my + 1 + j) % N_DEV
            rdma = pltpu.make_async_remote_copy(
                src_ref=ogather.at[pl.ds(my * SLICE, SLICE)],
                dst_ref=ogather.at[pl.ds(my * SLICE, SLICE)],
                send_sem=s2_sems.at[j],
                recv_sem=r2_sems.at[my],
                device_id=(peer,),
                device_id_type=pl.DeviceIdType.MESH,
            )
            rdma.start()
            sends.append(rdma)

        for j in range(N_DEV - 1) if not _NOCOMM else []:
            src = (my - 1 - j) % N_DEV
            recv = pltpu.make_async_remote_copy(
                src_ref=ogather.at[pl.ds(src * SLICE, SLICE)],
                dst_ref=ogather.at[pl.ds(src * SLICE, SLICE)],
                send_sem=s2_sems.at[j],
                recv_sem=r2_sems.at[src],
                device_id=(my,),
                device_id_type=pl.DeviceIdType.MESH,
            )
            recv.wait_recv()

        out_ref[:, :, :] = ogather[:, :].astype(jnp.float32).reshape(B, SQ, D)

        for rdma in sends:
            rdma.wait_send()

    return pl.pallas_call(
        body,
        out_shape=jax.ShapeDtypeStruct((B, SQ, D), jnp.float32),
        in_specs=[
            pl.BlockSpec(memory_space=pltpu.VMEM),
            pl.BlockSpec(memory_space=pltpu.MemorySpace.HBM),
            pl.BlockSpec(memory_space=pltpu.MemorySpace.HBM),
            pl.BlockSpec(memory_space=pltpu.VMEM),
            pl.BlockSpec(memory_space=pltpu.VMEM),
        ],
        out_specs=pl.BlockSpec(memory_space=pltpu.VMEM),
        scratch_shapes=[
            pltpu.VMEM((QR, PW), jnp.bfloat16),
            pltpu.VMEM((QR, PW), jnp.bfloat16),
            pltpu.VMEM((QR, D), jnp.bfloat16),
            pltpu.VMEM((D, D), jnp.float32),
            pltpu.VMEM((D, D), jnp.float32),
            pltpu.SemaphoreType.DMA((N_DEV - 1,)),
            pltpu.SemaphoreType.DMA((N_DEV,)),
            pltpu.SemaphoreType.DMA((N_DEV - 1,)),
            pltpu.SemaphoreType.DMA((N_DEV,)),
            pltpu.SemaphoreType.DMA((2,)),
        ],
        compiler_params=pltpu.CompilerParams(collective_id=0),
    )(x, Wq, Wo, k3, v3)


# device time: 17742 ns/iter; 1.0559x vs baseline; 1.0559x over previous
import os

import jax
import jax.numpy as jnp
from jax import lax
from jax.experimental import pallas as pl
from jax.experimental.pallas import tpu as pltpu

_NOCOMM = os.environ.get("NOCOMM") == "1"

N_DEV = 8
B, SQ, D = 2, 128, 512
HQ, HKV, DH = 8, 2, 64
GROUP = HQ // HKV
SKV_SH = 128
LANES = HKV * DH
QR = B * SQ
SLICE = QR // N_DEV
PW = HQ * DH + LANES


def kernel(x, Wq, Wo, K_ext, V_ext):
    def body(x_ref, wq_ref, wo_ref, k_ref, v_ref, out_ref,
             part_s, pbuf, ogather, s1_sems, r1_sems, s2_sems, r2_sems):
        my = lax.axis_index("i")

        barrier = pltpu.get_barrier_semaphore()
        for j in range(N_DEV - 1):
            peer = (my + 1 + j) % N_DEV
            pl.semaphore_signal(
                barrier, inc=1, device_id=(peer,),
                device_id_type=pl.DeviceIdType.MESH,
            )

        wq_bf = wq_ref[:, :].astype(jnp.bfloat16)
        qs = []
        for b in range(B):
            qb = jnp.dot(x_ref[b].astype(jnp.bfloat16), wq_bf,
                         preferred_element_type=jnp.float32)
            qs.append(qb * (0.125 * 1.4426950408889634))

        o_parts, l_parts = [], []
        for b in range(B):
            for kh in range(HKV):
                qg = jnp.concatenate(
                    [qs[b][:, (kh * GROUP + g) * DH:(kh * GROUP + g + 1) * DH]
                     for g in range(GROUP)], axis=0).astype(jnp.bfloat16)
                kc = k_ref[b, :, kh, :].astype(jnp.bfloat16)
                vc = v_ref[b, :, kh, :].astype(jnp.bfloat16)
                s_mat = lax.dot_general(
                    qg, kc, (((1,), (1,)), ((), ())),
                    preferred_element_type=jnp.float32)
                p = jnp.exp2(s_mat)
                l_parts.append(jnp.sum(p, axis=1, keepdims=True))
                o_parts.append(lax.dot_general(
                    p.astype(jnp.bfloat16), vc, (((1,), (0,)), ((), ())),
                    preferred_element_type=jnp.float32))

        rows = []
        for b in range(B):
            ob = jnp.concatenate(
                [o_parts[b * HKV + kh][g * SQ:(g + 1) * SQ, :]
                 for kh in range(HKV) for g in range(GROUP)], axis=1)
            lb = jnp.concatenate(
                [l_parts[b * HKV + kh][g * SQ:(g + 1) * SQ, :]
                 for kh in range(HKV) for g in range(GROUP)]
                + [jnp.zeros((SQ, LANES - HQ), jnp.float32)], axis=1)
            rows.append(jnp.concatenate([ob, lb], axis=1))
        part_s[:, :] = jnp.concatenate(rows, axis=0).astype(jnp.bfloat16)

        pbuf[pl.ds(my * SLICE, SLICE), :] = part_s[pl.ds(my * SLICE, SLICE), :]

        pl.semaphore_wait(barrier, N_DEV - 1)

        sends = []
        for j in range(N_DEV - 1) if not _NOCOMM else []:
            peer = (my + 1 + j) % N_DEV
            rdma = pltpu.make_async_remote_copy(
                src_ref=part_s.at[pl.ds(peer * SLICE, SLICE)],
                dst_ref=pbuf.at[pl.ds(my * SLICE, SLICE)],
                send_sem=s1_sems.at[j],
                recv_sem=r1_sems.at[my],
                device_id=(peer,),
                device_id_type=pl.DeviceIdType.MESH,
            )
            rdma.start()
            sends.append(rdma)

        acc = pbuf[pl.ds(my * SLICE, SLICE), :].astype(jnp.float32)
        for j in range(N_DEV - 1) if not _NOCOMM else []:
            src = (my - 1 - j) % N_DEV
            recv = pltpu.make_async_remote_copy(
                src_ref=part_s.at[pl.ds(src * SLICE, SLICE)],
                dst_ref=pbuf.at[pl.ds(src * SLICE, SLICE)],
                send_sem=s1_sems.at[j],
                recv_sem=r1_sems.at[src],
                device_id=(my,),
                device_id_type=pl.DeviceIdType.MESH,
            )
            recv.wait_recv()
            acc = acc + pbuf[pl.ds(src * SLICE, SLICE), :].astype(jnp.float32)
        oh = jnp.concatenate(
            [acc[:, h * DH:(h + 1) * DH] / acc[:, HQ * DH + h:HQ * DH + h + 1]
             for h in range(HQ)], axis=1)
        out_slice = jnp.dot(oh.astype(jnp.bfloat16),
                            wo_ref[:, :].astype(jnp.bfloat16),
                            preferred_element_type=jnp.float32)
        ogather[pl.ds(my * SLICE, SLICE), :] = out_slice.astype(jnp.bfloat16)

        for j in range(N_DEV - 1) if not _NOCOMM else []:
            peer = (my + 1 + j) % N_DEV
            rdma = pltpu.make_async_remote_copy(
                src_ref=ogather.at[pl.ds(my * SLICE, SLICE)],
                dst_ref=ogather.at[pl.ds(my * SLICE, SLICE)],
                send_sem=s2_sems.at[j],
                recv_sem=r2_sems.at[my],
                device_id=(peer,),
                device_id_type=pl.DeviceIdType.MESH,
            )
            rdma.start()
            sends.append(rdma)

        for j in range(N_DEV - 1) if not _NOCOMM else []:
            src = (my - 1 - j) % N_DEV
            recv = pltpu.make_async_remote_copy(
                src_ref=ogather.at[pl.ds(src * SLICE, SLICE)],
                dst_ref=ogather.at[pl.ds(src * SLICE, SLICE)],
                send_sem=s2_sems.at[j],
                recv_sem=r2_sems.at[src],
                device_id=(my,),
                device_id_type=pl.DeviceIdType.MESH,
            )
            recv.wait_recv()

        out_ref[:, :, :] = ogather[:, :].astype(jnp.float32).reshape(B, SQ, D)

        for rdma in sends:
            rdma.wait_send()

    return pl.pallas_call(
        body,
        out_shape=jax.ShapeDtypeStruct((B, SQ, D), jnp.float32),
        in_specs=[pl.BlockSpec(memory_space=pltpu.VMEM)] * 5,
        out_specs=pl.BlockSpec(memory_space=pltpu.VMEM),
        scratch_shapes=[
            pltpu.VMEM((QR, PW), jnp.bfloat16),
            pltpu.VMEM((QR, PW), jnp.bfloat16),
            pltpu.VMEM((QR, D), jnp.bfloat16),
            pltpu.SemaphoreType.DMA((N_DEV - 1,)),
            pltpu.SemaphoreType.DMA((N_DEV,)),
            pltpu.SemaphoreType.DMA((N_DEV - 1,)),
            pltpu.SemaphoreType.DMA((N_DEV,)),
        ],
        compiler_params=pltpu.CompilerParams(collective_id=0),
    )(x, Wq, Wo, K_ext, V_ext)
